# baseline (speedup 1.0000x reference)
import functools

import jax
import jax.numpy as jnp
from jax import lax
from jax.experimental import pallas as pl
from jax.experimental.pallas import tpu as pltpu
from jax.experimental.pallas import tpu_sc as plsc

B, N, CIN, C = 32, 10000, 4, 128
K = N // 2
KP = 5120
NV = N // 16
HV = NV // 2
E0 = HV * 16
CHUNK = 512
_NSC = 2


def _conv1x1(t, W, b):
    return jnp.einsum('bcnw,oc->bonw', t, W) + b[None, :, None, None]


def _inorm(t, eps=1e-3):
    m = t.mean(axis=(2, 3), keepdims=True)
    v = t.var(axis=(2, 3), keepdims=True)
    return (t - m) / jnp.sqrt(v + eps)


def _bnorm(t, eps=1e-5):
    m = t.mean(axis=(0, 2, 3), keepdims=True)
    v = t.var(axis=(0, 2, 3), keepdims=True)
    return (t - m) / jnp.sqrt(v + eps)


def _scores(x, W_in, b_in, W1, b1, W2, b2, W0, b0):
    out = jnp.transpose(x, (0, 3, 2, 1))
    out = _conv1x1(out, W_in, b_in)
    left = _conv1x1(out, W1, b1)
    left = jax.nn.relu(_bnorm(_inorm(left)))
    left = _conv1x1(left, W2, b2)
    left = _bnorm(_inorm(left))
    out = jax.nn.relu(left + out)
    return _conv1x1(out, W0, b0).reshape(x.shape[0], x.shape[2])


import numpy as _np

_TOP = _np.uint32(0x80000000)
_LOW = _np.uint32(0x7FFFFFFF)


def _flip(fv):
    bits = plsc.bitcast(fv, jnp.uint32)
    neg = (bits & _TOP) != 0
    return plsc.bitcast(jnp.where(neg, bits, ~(bits | _TOP)), jnp.int32)


def _unflip(ki):
    key = plsc.bitcast(ki, jnp.uint32)
    neg = (key & _TOP) != 0
    return plsc.bitcast(jnp.where(neg, key, (~key) & _LOW), jnp.float32)


@functools.partial(
    pl.kernel,
    mesh=plsc.VectorSubcoreMesh(core_axis_name="c", subcore_axis_name="s"),
    compiler_params=pltpu.CompilerParams(needs_layout_passes=False),
    out_type=[jax.ShapeDtypeStruct((B, KP), jnp.float32),
              jax.ShapeDtypeStruct((B * CIN, KP), jnp.float32),
              jax.ShapeDtypeStruct((B * 2, KP), jnp.float32)],
    scratch_types=[pltpu.VMEM((N,), jnp.float32),
                   pltpu.VMEM((N,), jnp.int32),
                   pltpu.VMEM((N,), jnp.int32),
                   pltpu.VMEM((N,), jnp.int32),
                   pltpu.VMEM((N,), jnp.int32),
                   pltpu.VMEM((N * CIN,), jnp.float32),
                   pltpu.VMEM((N * 2,), jnp.float32),
                   pltpu.VMEM((2048,), jnp.int32),
                   pltpu.VMEM((2048,), jnp.int32),
                   pltpu.VMEM((2048,), jnp.int32),
                   pltpu.VMEM((2048,), jnp.int32),
                   pltpu.VMEM((2, CHUNK), jnp.float32),
                   pltpu.VMEM((2, CIN, CHUNK), jnp.float32),
                   pltpu.VMEM((2, 2, CHUNK), jnp.float32),
                   pltpu.SemaphoreType.DMA,
                   pltpu.SemaphoreType.DMA],
)
def _sc_topk_gather(lg_hbm, x_hbm, y_hbm, w_hbm, xo_hbm, yo_hbm,
                    lrow, kA, kB, iA, iB, xslab, yslab,
                    hist0, hist1, offs0, offs1, wst, xc, yc,
                    sem_in, sem_out):
    wid = lax.axis_index("s") * _NSC + lax.axis_index("c")
    cx = pltpu.async_copy(x_hbm.at[wid], xslab, sem_in)
    cy = pltpu.async_copy(y_hbm.at[wid], yslab, sem_in)
    pltpu.sync_copy(lg_hbm.at[wid], lrow)

    ones = jnp.ones((16,), jnp.int32)
    zeros = jnp.zeros((16,), jnp.int32)

    shifts = (0, 11, 22)

    def digit(kv, p):
        return (kv >> shifts[p]) & 0x7FF

    def zero_hists():
        @plsc.parallel_loop(0, 2048 // 16, unroll=4)
        def _(j):
            hist0[pl.ds(j * 16, 16)] = zeros
            hist1[pl.ds(j * 16, 16)] = zeros

    def scan_hists():
        @plsc.parallel_loop(0, 2048 // 16, unroll=2, carry=jnp.int32(0))
        def _(j, run):
            h0 = hist0[pl.ds(j * 16, 16)]
            h1 = hist1[pl.ds(j * 16, 16)]
            hv = h0 + h1
            inc = plsc.cumsum(hv)
            excl = inc - hv + run
            offs0[pl.ds(j * 16, 16)] = excl
            offs1[pl.ds(j * 16, 16)] = excl + h0
            return run + jnp.sum(hv)

    zero_hists()

    @plsc.parallel_loop(0, HV, unroll=4)
    def _(i):
        kv = _flip(lrow[pl.ds(i * 16, 16)])
        kA[pl.ds(i * 16, 16)] = kv
        plsc.addupdate_scatter(hist0, [digit(kv, 0)], ones)

    @plsc.parallel_loop(HV, NV)
    def _(i):
        kv = _flip(lrow[pl.ds(i * 16, 16)])
        kA[pl.ds(i * 16, 16)] = kv
        plsc.addupdate_scatter(hist1, [digit(kv, 0)], ones)

    for p, (src_k, src_i, dst_k, dst_i) in enumerate(
            [(kA, None, kB, iB), (kB, iB, kA, iA), (kA, iA, kB, iB)]):
        scan_hists()
        if p < 2:
            zero_hists()

        def perm_one(i, offs, p=p, src_k=src_k, src_i=src_i,
                     dst_k=dst_k, dst_i=dst_i):
            kv = src_k[pl.ds(i * 16, 16)]
            d = digit(kv, p)
            if p == 0:
                iv = i * 16 + lax.iota(jnp.int32, 16)
            else:
                iv = src_i[pl.ds(i * 16, 16)]
            base = plsc.load_gather(offs, [d])
            cnt, _ = plsc.scan_count(d)
            slot = base + cnt - 1
            plsc.store_scatter(dst_k, [slot], kv)
            plsc.store_scatter(dst_i, [slot], iv)
            plsc.store_scatter(offs, [d], base + cnt)
            if p < 2:
                dn = digit(kv, p + 1)
                m0 = slot < E0
                plsc.addupdate_scatter(hist0, [dn], ones, mask=m0)
                plsc.addupdate_scatter(hist1, [dn], ones,
                                       mask=jnp.logical_not(m0))

        def perm_body(i2, carry, perm_one=perm_one):
            for u in range(2):
                i = i2 * 2 + u
                perm_one(i, offs0)
                perm_one(HV + i, offs1)
            return carry

        lax.fori_loop(0, HV // 2, perm_body, 0)
        perm_one(NV - 1, offs1)

    cx.wait()
    cy.wait()
    nch = KP // CHUNK

    def chunk_copies(c, buf):
        yield (wst.at[buf], w_hbm.at[wid, pl.ds(c * CHUNK, CHUNK)])
        for cc in range(CIN):
            yield (xc.at[buf, cc],
                   xo_hbm.at[wid * CIN + cc, pl.ds(c * CHUNK, CHUNK)])
        for cc in range(2):
            yield (yc.at[buf, cc],
                   yo_hbm.at[wid * 2 + cc, pl.ds(c * CHUNK, CHUNK)])

    for c in range(nch):
        buf = c % 2
        if c >= 2:
            for src, dst in chunk_copies(c - 2, buf):
                pltpu.make_async_copy(src, dst, sem_out).wait()

        @plsc.parallel_loop(0, CHUNK // 16, unroll=2)
        def _(jj, c=c, buf=buf):
            t = c * (CHUNK // 16) + jj
            idx16 = iB[pl.ds(t * 16, 16)]
            wst[buf, pl.ds(jj * 16, 16)] = _unflip(kB[pl.ds(t * 16, 16)])
            for cc in range(CIN):
                xc[buf, cc, pl.ds(jj * 16, 16)] = plsc.load_gather(
                    xslab, [idx16 * CIN + cc])
            for cc in range(2):
                yc[buf, cc, pl.ds(jj * 16, 16)] = plsc.load_gather(
                    yslab, [idx16 * 2 + cc])
        for src, dst in chunk_copies(c, buf):
            pltpu.async_copy(src, dst, sem_out)

    for c in range(nch - 2, nch):
        for src, dst in chunk_copies(c, c % 2):
            pltpu.make_async_copy(src, dst, sem_out).wait()


def kernel(x, y, W_in, b_in, W1, b1, W2, b2, W0, b0):
    logits = _scores(x, W_in, b_in, W1, b1, W2, b2, W0, b0)
    w_pad, xg, yg = _sc_topk_gather(
        logits, x.reshape(B, N * CIN), y.reshape(B, N * 2))
    x_ds = xg.reshape(B, CIN, KP)[:, :, :K].transpose(0, 2, 1)
    y_ds = yg.reshape(B, 2, KP)[:, :, :K].transpose(0, 2, 1)
    return (x_ds.reshape(B, 1, K, CIN), y_ds.reshape(B, 1, K, 2),
            w_pad[:, :K])


# speedup vs baseline: 1.0053x; 1.0053x over previous
import functools

import jax
import jax.numpy as jnp
from jax import lax
from jax.experimental import pallas as pl
from jax.experimental.pallas import tpu as pltpu
from jax.experimental.pallas import tpu_sc as plsc

B, N, CIN, C = 32, 10000, 4, 128
K = N // 2
KP = 5120
NV = N // 16
CHUNK = 1024
_NSC = 2


def _conv1x1(t, W, b):
    return jnp.einsum('bcnw,oc->bonw', t, W) + b[None, :, None, None]


def _inorm(t, eps=1e-3):
    m = t.mean(axis=(2, 3), keepdims=True)
    v = t.var(axis=(2, 3), keepdims=True)
    return (t - m) / jnp.sqrt(v + eps)


def _bnorm(t, eps=1e-5):
    m = t.mean(axis=(0, 2, 3), keepdims=True)
    v = t.var(axis=(0, 2, 3), keepdims=True)
    return (t - m) / jnp.sqrt(v + eps)


def _scores(x, W_in, b_in, W1, b1, W2, b2, W0, b0):
    out = jnp.transpose(x, (0, 3, 2, 1))
    out = _conv1x1(out, W_in, b_in)
    left = _conv1x1(out, W1, b1)
    left = jax.nn.relu(_bnorm(_inorm(left)))
    left = _conv1x1(left, W2, b2)
    left = _bnorm(_inorm(left))
    out = jax.nn.relu(left + out)
    return _conv1x1(out, W0, b0).reshape(x.shape[0], x.shape[2])


import numpy as _np

_TOP = _np.uint32(0x80000000)
_LOW = _np.uint32(0x7FFFFFFF)


def _flip(fv):
    bits = plsc.bitcast(fv, jnp.uint32)
    neg = (bits & _TOP) != 0
    return plsc.bitcast(jnp.where(neg, bits, ~(bits | _TOP)), jnp.int32)


def _unflip(ki):
    key = plsc.bitcast(ki, jnp.uint32)
    neg = (key & _TOP) != 0
    return plsc.bitcast(jnp.where(neg, key, (~key) & _LOW), jnp.float32)


@functools.partial(
    pl.kernel,
    mesh=plsc.VectorSubcoreMesh(core_axis_name="c", subcore_axis_name="s"),
    compiler_params=pltpu.CompilerParams(needs_layout_passes=False),
    out_type=[jax.ShapeDtypeStruct((B, KP), jnp.float32),
              jax.ShapeDtypeStruct((B * CIN, KP), jnp.float32),
              jax.ShapeDtypeStruct((B * 2, KP), jnp.float32)],
    scratch_types=[pltpu.VMEM((N,), jnp.float32),
                   pltpu.VMEM((N,), jnp.int32),
                   pltpu.VMEM((N,), jnp.int32),
                   pltpu.VMEM((N,), jnp.int32),
                   pltpu.VMEM((N,), jnp.int32),
                   pltpu.VMEM((N * CIN,), jnp.float32),
                   pltpu.VMEM((N * 2,), jnp.float32),
                   pltpu.VMEM((2048,), jnp.int32),
                   pltpu.VMEM((2048,), jnp.int32),
                   pltpu.VMEM((2, CHUNK), jnp.float32),
                   pltpu.VMEM((2, CIN, CHUNK), jnp.float32),
                   pltpu.VMEM((2, 2, CHUNK), jnp.float32),
                   pltpu.SemaphoreType.DMA,
                   pltpu.SemaphoreType.DMA],
)
def _sc_topk_gather(lg_hbm, x_hbm, y_hbm, w_hbm, xo_hbm, yo_hbm,
                    lrow, kA, kB, iA, iB, xslab, yslab,
                    hist, offs, wst, xc, yc, sem_in, sem_out):
    wid = lax.axis_index("s") * _NSC + lax.axis_index("c")
    cx = pltpu.async_copy(x_hbm.at[wid], xslab, sem_in)
    cy = pltpu.async_copy(y_hbm.at[wid], yslab, sem_in)
    pltpu.sync_copy(lg_hbm.at[wid], lrow)

    ones = jnp.ones((16,), jnp.int32)
    zeros = jnp.zeros((16,), jnp.int32)

    shifts = (0, 11, 22)

    def digit(kv, p):
        return (kv >> shifts[p]) & 0x7FF

    def zero_hist():
        @plsc.parallel_loop(0, 2048 // 16, unroll=4)
        def _(j):
            hist[pl.ds(j * 16, 16)] = zeros

    def scan_hist():
        @plsc.parallel_loop(0, 2048 // 16, unroll=2, carry=jnp.int32(0))
        def _(j, run):
            hv = hist[pl.ds(j * 16, 16)]
            inc = plsc.cumsum(hv)
            offs[pl.ds(j * 16, 16)] = inc - hv + run
            return run + jnp.sum(hv)

    zero_hist()

    @plsc.parallel_loop(0, NV, unroll=4)
    def _(i):
        kv = _flip(lrow[pl.ds(i * 16, 16)])
        kA[pl.ds(i * 16, 16)] = kv
        plsc.addupdate_scatter(hist, [digit(kv, 0)], ones)

    for p, (src_k, src_i, dst_k, dst_i) in enumerate(
            [(kA, None, kB, iB), (kB, iB, kA, iA), (kA, iA, kB, iB)]):
        scan_hist()
        if p < 2:
            zero_hist()

        def perm_body(i5, carry, p=p, src_k=src_k, src_i=src_i,
                      dst_k=dst_k, dst_i=dst_i):
            for u in range(5):
                i = i5 * 5 + u
                kv = src_k[pl.ds(i * 16, 16)]
                d = digit(kv, p)
                if p == 0:
                    iv = i * 16 + lax.iota(jnp.int32, 16)
                else:
                    iv = src_i[pl.ds(i * 16, 16)]
                base = plsc.load_gather(offs, [d])
                cnt, _ = plsc.scan_count(d)
                slot = base + cnt - 1
                plsc.store_scatter(dst_k, [slot], kv)
                plsc.store_scatter(dst_i, [slot], iv)
                plsc.store_scatter(offs, [d], base + cnt)
                if p < 2:
                    plsc.addupdate_scatter(hist, [digit(kv, p + 1)], ones)
            return carry

        lax.fori_loop(0, NV // 5, perm_body, 0)

    cx.wait()
    cy.wait()
    nch = KP // CHUNK

    def chunk_copies(c, buf):
        yield (wst.at[buf], w_hbm.at[wid, pl.ds(c * CHUNK, CHUNK)])
        for cc in range(CIN):
            yield (xc.at[buf, cc],
                   xo_hbm.at[wid * CIN + cc, pl.ds(c * CHUNK, CHUNK)])
        for cc in range(2):
            yield (yc.at[buf, cc],
                   yo_hbm.at[wid * 2 + cc, pl.ds(c * CHUNK, CHUNK)])

    for c in range(nch):
        buf = c % 2
        if c >= 2:
            for src, dst in chunk_copies(c - 2, buf):
                pltpu.make_async_copy(src, dst, sem_out).wait()

        @plsc.parallel_loop(0, CHUNK // 16, unroll=2)
        def _(jj, c=c, buf=buf):
            t = c * (CHUNK // 16) + jj
            idx16 = iB[pl.ds(t * 16, 16)]
            wst[buf, pl.ds(jj * 16, 16)] = _unflip(kB[pl.ds(t * 16, 16)])
            for cc in range(CIN):
                xc[buf, cc, pl.ds(jj * 16, 16)] = plsc.load_gather(
                    xslab, [idx16 * CIN + cc])
            for cc in range(2):
                yc[buf, cc, pl.ds(jj * 16, 16)] = plsc.load_gather(
                    yslab, [idx16 * 2 + cc])
        for src, dst in chunk_copies(c, buf):
            pltpu.async_copy(src, dst, sem_out)

    for c in range(nch - 2, nch):
        for src, dst in chunk_copies(c, c % 2):
            pltpu.make_async_copy(src, dst, sem_out).wait()


def kernel(x, y, W_in, b_in, W1, b1, W2, b2, W0, b0):
    logits = _scores(x, W_in, b_in, W1, b1, W2, b2, W0, b0)
    w_pad, xg, yg = _sc_topk_gather(
        logits, x.reshape(B, N * CIN), y.reshape(B, N * 2))
    x_ds = xg.reshape(B, CIN, KP)[:, :, :K].transpose(0, 2, 1)
    y_ds = yg.reshape(B, 2, KP)[:, :, :K].transpose(0, 2, 1)
    return (x_ds.reshape(B, 1, K, CIN), y_ds.reshape(B, 1, K, 2),
            w_pad[:, :K])
